# baseline (speedup 1.0000x reference)
import functools

import jax
import jax.numpy as jnp
from jax import lax
from jax.experimental import pallas as pl
from jax.experimental.pallas import tpu as pltpu
from jax.experimental.pallas import tpu_sc as plsc

N_NODES = 10000
HID = 128
D_IN = 256
EDGES = 320000
_K = 64
_NT = 16
_ACC_N = 10240
_JUNK = _ACC_N - N_NODES
_CW = 16


def _make_segsum(n_win: int):
    nc = n_win // _CW
    assert n_win % _CW == 0 and nc >= 3
    rpt = 624
    left = N_NODES - _NT * rpt

    mesh = plsc.VectorSubcoreMesh(core_axis_name="c", subcore_axis_name="s")

    @functools.partial(
        pl.kernel,
        out_type=jax.ShapeDtypeStruct((2, N_NODES, HID), jnp.float32),
        mesh=mesh,
        scratch_types=[
            pltpu.VMEM_SHARED((_ACC_N, HID), jnp.float32),
            pltpu.VMEM((16, 2 * _K), jnp.int32),
            pltpu.VMEM((16, 2 * _K), jnp.int32),
            pltpu.VMEM((_K, HID), jnp.float32),
            pltpu.VMEM((_K, HID), jnp.float32),
            pltpu.VMEM((_K, HID), jnp.float32),
            pltpu.VMEM((_K, HID), jnp.float32),
            pltpu.SemaphoreType.DMA,
            pltpu.SemaphoreType.DMA,
            pltpu.SemaphoreType.DMA,
            pltpu.SemaphoreType.DMA,
            pltpu.SemaphoreType.DMA,
            pltpu.SemaphoreType.DMA,
            pltpu.SemaphoreType.DMA,
            pltpu.SemaphoreType.DMA,
            pltpu.SemaphoreType.DMA,
            pltpu.SemaphoreType.DMA,
        ],
    )
    def seg(table_hbm, src_hbm, dst_hbm, out_hbm, acc, idx_s, idx_d,
            r0, r1, r2, r3, g0, g1, g2, g3, s0, s1, s2, s3, li_s, li_d):
        c = lax.axis_index("c")
        s = lax.axis_index("s")
        rows = [r0, r1, r2, r3]
        semg = [g0, g1, g2, g3]
        sems = [s0, s1, s2, s3]
        base = (c * _NT + s) * (n_win // 2)

        def load_chunk(k, rb):
            pltpu.async_copy(src_hbm.at[pl.ds(base + k * 8, 8)],
                             idx_s.at[pl.ds(rb, 8)], li_s)
            pltpu.async_copy(dst_hbm.at[pl.ds(base + k * 8, 8)],
                             idx_d.at[pl.ds(rb, 8)], li_d)

        def wait_chunk(k, rb):
            pltpu.make_async_copy(src_hbm.at[pl.ds(base + k * 8, 8)],
                                  idx_s.at[pl.ds(rb, 8)], li_s).wait()
            pltpu.make_async_copy(dst_hbm.at[pl.ds(base + k * 8, 8)],
                                  idx_d.at[pl.ds(rb, 8)], li_d).wait()

        load_chunk(0, 0)
        zero = jnp.zeros((16,), jnp.float32)

        def zrow(r, carry):
            for k in range(HID // 16):
                r0[r, pl.ds(k * 16, 16)] = zero
            return carry

        lax.fori_loop(0, _K, zrow, 0)
        for j in range(rpt // _K):
            pltpu.sync_copy(r0, acc.at[pl.ds(s * rpt + j * _K, _K)])
        pltpu.sync_copy(r0.at[pl.ds(0, rpt % _K)],
                        acc.at[pl.ds(s * rpt + (rpt // _K) * _K, rpt % _K)])

        @pl.when(s == 0)
        def _zero_tail():
            pltpu.sync_copy(r0.at[pl.ds(0, left)],
                            acc.at[pl.ds(_NT * rpt, left)])

        wait_chunk(0, 0)
        plsc.subcore_barrier()

        def gather(r, o, b):
            return pltpu.async_copy(table_hbm.at[idx_s.at[r, pl.ds(o, _K)]],
                                    rows[b], semg[b])

        def scat(r, o, b):
            return pltpu.async_copy(rows[b], acc.at[idx_d.at[r, pl.ds(o, _K)]],
                                    sems[b], add=True)

        def wait_g(r, o, b):
            pltpu.make_async_copy(table_hbm.at[idx_s.at[r, pl.ds(o, _K)]],
                                  rows[b], semg[b]).wait()

        def wait_s(r, o, b):
            pltpu.make_async_copy(rows[b], acc.at[idx_d.at[r, pl.ds(o, _K)]],
                                  sems[b]).wait()

        _o = lambda j: (j % 2) * _K

        def pair(rb, rbp, rbn, j2, last):
            b0 = (2 * j2) % 4
            b2 = (b0 + 2) % 4
            if j2 == 0:
                wait_s(rbp + 7, _o(0), b2)
                wait_s(rbp + 7, _o(1), b2 + 1)
            else:
                wait_s(rb + j2 - 1, _o(0), b2)
                wait_s(rb + j2 - 1, _o(1), b2 + 1)
            if not last:
                if j2 == 7:
                    gather(rbn, _o(0), b2)
                    gather(rbn, _o(1), b2 + 1)
                else:
                    gather(rb + j2 + 1, _o(0), b2)
                    gather(rb + j2 + 1, _o(1), b2 + 1)
            wait_g(rb + j2, _o(0), b0)
            scat(rb + j2, _o(0), b0)
            wait_g(rb + j2, _o(1), b0 + 1)
            scat(rb + j2, _o(1), b0 + 1)

        gather(0, _o(0), 0)
        gather(0, _o(1), 1)
        gather(1, _o(0), 2)
        gather(1, _o(1), 3)
        wait_g(0, _o(0), 0)
        scat(0, _o(0), 0)
        wait_g(0, _o(1), 1)
        scat(0, _o(1), 1)
        load_chunk(1, 8)
        for j2 in range(1, 7):
            pair(0, None, 8, j2, last=False)
        wait_chunk(1, 8)
        pair(0, None, 8, 7, last=False)

        def chunk_body(cc, carry):
            rb = lax.rem(cc, 2) * 8
            rbn = lax.rem(cc + 1, 2) * 8
            pair(rb, rbn, rbn, 0, last=False)
            load_chunk(cc + 1, rbn)
            for j2 in range(1, 7):
                pair(rb, rbn, rbn, j2, last=False)
            wait_chunk(cc + 1, rbn)
            pair(rb, rbn, rbn, 7, last=False)
            return carry

        lax.fori_loop(1, nc - 1, chunk_body, 0)

        rb = ((nc - 1) % 2) * 8
        rbp = (nc % 2) * 8
        for j2 in range(7):
            pair(rb, rbp, rbp, j2, last=False)
        pair(rb, rbp, rbp, 7, last=True)
        wait_s(rb + 7, _o(0), 2)
        wait_s(rb + 7, _o(1), 3)

        plsc.subcore_barrier()

        pltpu.sync_copy(acc.at[pl.ds(s * rpt, rpt)],
                        out_hbm.at[c, pl.ds(s * rpt, rpt)])

        @pl.when(s == 0)
        def _out_tail():
            pltpu.sync_copy(acc.at[pl.ds(_NT * rpt, left)],
                            out_hbm.at[c, pl.ds(_NT * rpt, left)])

    return seg


_seg_half = _make_segsum(160)


def _window_idx(src, dst, n_win, table_rows):
    ept = src.shape[1] // _NT
    pad = n_win * _K - ept
    src_r = src.reshape(2, _NT, ept)
    dst_r = dst.reshape(2, _NT, ept)
    ar = jnp.arange(pad, dtype=jnp.int32)
    pad_src = jnp.broadcast_to(ar % table_rows, (2, _NT, pad))
    pad_dst = jnp.broadcast_to(N_NODES + ar % _JUNK, (2, _NT, pad))
    src_w = jnp.concatenate([src_r, pad_src], axis=2).reshape(-1, 2 * _K)
    dst_w = jnp.concatenate([dst_r, pad_dst], axis=2).reshape(-1, 2 * _K)
    return src_w, dst_w


_BP = 2000


def _proj_body(x_ref, w_ref, b_ref, o_ref):
    y = jnp.dot(x_ref[...], w_ref[0], preferred_element_type=jnp.float32)
    o_ref[...] = jnp.maximum(y + b_ref[0], 0.0)


def _proj(x_cat, w_stack, b_stack):
    n = x_cat.shape[0]
    grid = (n // _BP,)
    sel = lambda i: (i * _BP) // N_NODES
    return pl.pallas_call(
        _proj_body,
        grid=grid,
        in_specs=[
            pl.BlockSpec((_BP, D_IN), lambda i: (i, 0)),
            pl.BlockSpec((1, D_IN, HID), lambda i: (sel(i), 0, 0)),
            pl.BlockSpec((1, 1, HID), lambda i: (sel(i), 0, 0)),
        ],
        out_specs=pl.BlockSpec((_BP, HID), lambda i: (i, 0)),
        out_shape=jax.ShapeDtypeStruct((n, HID), jnp.float32),
    )(x_cat, w_stack, b_stack)


def _mlp2_body(p_ref, h_ref, w1_ref, b1_ref, w2_ref, b2_ref, o_ref):
    y = p_ref[0] + p_ref[1] + h_ref[...]
    t = jnp.dot(y, w1_ref[...], preferred_element_type=jnp.float32)
    t = jnp.maximum(t + b1_ref[...], 0.0)
    o = jnp.dot(t, w2_ref[...], preferred_element_type=jnp.float32)
    o_ref[...] = jnp.maximum(o + b2_ref[...], 0.0)


def _mlp2(parts, h_cat, off, w1, b1, w2, b2):
    grid = (N_NODES // _BP,)
    return pl.pallas_call(
        _mlp2_body,
        grid=grid,
        in_specs=[
            pl.BlockSpec((2, _BP, HID), lambda i: (0, i, 0)),
            pl.BlockSpec((_BP, HID), lambda i: (i + off, 0)),
            pl.BlockSpec((HID, HID), lambda i: (0, 0)),
            pl.BlockSpec((1, HID), lambda i: (0, 0)),
            pl.BlockSpec((HID, HID), lambda i: (0, 0)),
            pl.BlockSpec((1, HID), lambda i: (0, 0)),
        ],
        out_specs=pl.BlockSpec((_BP, HID), lambda i: (i, 0)),
        out_shape=jax.ShapeDtypeStruct((N_NODES, HID), jnp.float32),
    )(parts, h_cat, w1, b1, w2, b2)


def _final_body(p_ref, h_ref, w1_ref, b1_ref, w2_ref, b2_ref, wo_ref, bo_ref,
                o_ref):
    y = p_ref[0] + p_ref[1] + h_ref[...]
    t = jnp.dot(y, w1_ref[...], preferred_element_type=jnp.float32)
    t = jnp.maximum(t + b1_ref[...], 0.0)
    h2 = jnp.dot(t, w2_ref[...], preferred_element_type=jnp.float32)
    h2 = jnp.maximum(h2 + b2_ref[...], 0.0)
    o_ref[...] = jnp.dot(h2, wo_ref[...],
                         preferred_element_type=jnp.float32) + bo_ref[...]


def _final(parts, h_p, w1, b1, w2, b2, wo_pad, bo_pad):
    grid = (N_NODES // _BP,)
    return pl.pallas_call(
        _final_body,
        grid=grid,
        in_specs=[
            pl.BlockSpec((2, _BP, HID), lambda i: (0, i, 0)),
            pl.BlockSpec((_BP, HID), lambda i: (i, 0)),
            pl.BlockSpec((HID, HID), lambda i: (0, 0)),
            pl.BlockSpec((1, HID), lambda i: (0, 0)),
            pl.BlockSpec((HID, HID), lambda i: (0, 0)),
            pl.BlockSpec((1, HID), lambda i: (0, 0)),
            pl.BlockSpec((HID, HID), lambda i: (0, 0)),
            pl.BlockSpec((1, HID), lambda i: (0, 0)),
        ],
        out_specs=pl.BlockSpec((_BP, HID), lambda i: (i, 0)),
        out_shape=jax.ShapeDtypeStruct((N_NODES, HID), jnp.float32),
    )(parts, h_p, w1, b1, w2, b2, wo_pad, bo_pad)


@jax.jit
def kernel(x_product, x_user, ei_buys, ei_rev, Wp_in, bp_in, Wu_in, bu_in,
           l0_buys_W1, l0_buys_b1, l0_buys_W2, l0_buys_b2,
           l0_rev_W1, l0_rev_b1, l0_rev_W2, l0_rev_b2,
           l1_buys_W1, l1_buys_b1, l1_buys_W2, l1_buys_b2,
           l1_rev_W1, l1_rev_b1, l1_rev_W2, l1_rev_b2,
           W_out, b_out):
    ei_b = jnp.asarray(ei_buys, jnp.int32)
    ei_r = jnp.asarray(ei_rev, jnp.int32)

    x_cat = jnp.concatenate([x_product, x_user], axis=0)
    w_in = jnp.stack([Wp_in, Wu_in])
    b_in = jnp.stack([bp_in, bu_in]).reshape(2, 1, HID)
    h_cat0 = _proj(x_cat, w_in, b_in)

    srcUw, dstUw = _window_idx(ei_r[0].reshape(2, EDGES // 2),
                               ei_r[1].reshape(2, EDGES // 2), 160,
                               2 * N_NODES)
    aggU = _seg_half(h_cat0, srcUw, dstUw)

    srcPw, dstPw = _window_idx(
        ei_b[0].reshape(2, EDGES // 2) + N_NODES,
        ei_b[1].reshape(2, EDGES // 2), 160, 2 * N_NODES)
    aggP = _seg_half(h_cat0, srcPw, dstPw)

    h_u1 = _mlp2(aggU, h_cat0, N_NODES // _BP,
                 l0_rev_W1, l0_rev_b1.reshape(1, HID),
                 l0_rev_W2, l0_rev_b2.reshape(1, HID))

    src1w, dst1w = _window_idx(ei_b[0].reshape(2, EDGES // 2),
                               ei_b[1].reshape(2, EDGES // 2), 160, N_NODES)
    parts = _seg_half(h_u1, src1w, dst1w)

    h_p1 = _mlp2(aggP, h_cat0, 0,
                 l0_buys_W1, l0_buys_b1.reshape(1, HID),
                 l0_buys_W2, l0_buys_b2.reshape(1, HID))

    wo_pad = jnp.zeros((HID, HID), jnp.float32).at[:, 0].set(W_out[:, 0])
    bo_pad = jnp.zeros((1, HID), jnp.float32).at[0, 0].set(b_out[0])
    out = _final(parts, h_p1, l1_buys_W1, l1_buys_b1.reshape(1, HID),
                 l1_buys_W2, l1_buys_b2.reshape(1, HID), wo_pad, bo_pad)
    return out[:, 0]


# speedup vs baseline: 1.0471x; 1.0471x over previous
import functools

import jax
import jax.numpy as jnp
from jax import lax
from jax.experimental import pallas as pl
from jax.experimental.pallas import tpu as pltpu
from jax.experimental.pallas import tpu_sc as plsc

N_NODES = 10000
HID = 128
D_IN = 256
EDGES = 320000
_K = 64
_NT = 16
_ACC_N = 10240
_JUNK = _ACC_N - N_NODES
_CW = 16


def _make_segsum(n_win: int):
    nc = n_win // _CW
    assert n_win % _CW == 0 and nc >= 3
    rpt = 624
    left = N_NODES - _NT * rpt

    mesh = plsc.VectorSubcoreMesh(core_axis_name="c", subcore_axis_name="s")

    @functools.partial(
        pl.kernel,
        out_type=jax.ShapeDtypeStruct((2, N_NODES, HID), jnp.float32),
        mesh=mesh,
        scratch_types=[
            pltpu.VMEM_SHARED((_ACC_N, HID), jnp.float32),
            pltpu.VMEM((16, 2 * _K), jnp.int32),
            pltpu.VMEM((16, 2 * _K), jnp.int32),
            pltpu.VMEM((_K, HID), jnp.float32),
            pltpu.VMEM((_K, HID), jnp.float32),
            pltpu.VMEM((_K, HID), jnp.float32),
            pltpu.VMEM((_K, HID), jnp.float32),
            pltpu.SemaphoreType.DMA,
            pltpu.SemaphoreType.DMA,
            pltpu.SemaphoreType.DMA,
            pltpu.SemaphoreType.DMA,
            pltpu.SemaphoreType.DMA,
            pltpu.SemaphoreType.DMA,
            pltpu.SemaphoreType.DMA,
            pltpu.SemaphoreType.DMA,
            pltpu.SemaphoreType.DMA,
            pltpu.SemaphoreType.DMA,
        ],
    )
    def seg(table_hbm, src_hbm, dst_hbm, out_hbm, acc, idx_s, idx_d,
            r0, r1, r2, r3, g0, g1, g2, g3, s0, s1, s2, s3, li_s, li_d):
        c = lax.axis_index("c")
        s = lax.axis_index("s")
        rows = [r0, r1, r2, r3]
        semg = [g0, g1, g2, g3]
        sems = [s0, s1, s2, s3]
        base = (c * _NT + s) * (n_win // 2)

        def load_chunk(k, rb):
            pltpu.async_copy(src_hbm.at[pl.ds(base + k * 8, 8)],
                             idx_s.at[pl.ds(rb, 8)], li_s)
            pltpu.async_copy(dst_hbm.at[pl.ds(base + k * 8, 8)],
                             idx_d.at[pl.ds(rb, 8)], li_d)

        def wait_chunk(k, rb):
            pltpu.make_async_copy(src_hbm.at[pl.ds(base + k * 8, 8)],
                                  idx_s.at[pl.ds(rb, 8)], li_s).wait()
            pltpu.make_async_copy(dst_hbm.at[pl.ds(base + k * 8, 8)],
                                  idx_d.at[pl.ds(rb, 8)], li_d).wait()

        load_chunk(0, 0)
        zero = jnp.zeros((16,), jnp.float32)

        def zrow(r, carry):
            for k in range(HID // 16):
                r0[r, pl.ds(k * 16, 16)] = zero
            return carry

        lax.fori_loop(0, _K, zrow, 0)
        for j in range(rpt // _K):
            pltpu.sync_copy(r0, acc.at[pl.ds(s * rpt + j * _K, _K)])
        pltpu.sync_copy(r0.at[pl.ds(0, rpt % _K)],
                        acc.at[pl.ds(s * rpt + (rpt // _K) * _K, rpt % _K)])

        @pl.when(s == 0)
        def _zero_tail():
            pltpu.sync_copy(r0.at[pl.ds(0, left)],
                            acc.at[pl.ds(_NT * rpt, left)])

        wait_chunk(0, 0)
        plsc.subcore_barrier()

        def gather(r, o, b):
            return pltpu.async_copy(table_hbm.at[idx_s.at[r, pl.ds(o, _K)]],
                                    rows[b], semg[b])

        def scat(r, o, b):
            return pltpu.async_copy(rows[b], acc.at[idx_d.at[r, pl.ds(o, _K)]],
                                    sems[b], add=True)

        def wait_g(r, o, b):
            pltpu.make_async_copy(table_hbm.at[idx_s.at[r, pl.ds(o, _K)]],
                                  rows[b], semg[b]).wait()

        def wait_s(r, o, b):
            pltpu.make_async_copy(rows[b], acc.at[idx_d.at[r, pl.ds(o, _K)]],
                                  sems[b]).wait()

        _o = lambda j: (j % 2) * _K

        def pair(rb, rbp, rbn, j2, last):
            b0 = (2 * j2) % 4
            b2 = (b0 + 2) % 4
            if j2 == 0:
                wait_s(rbp + 7, _o(0), b2)
                wait_s(rbp + 7, _o(1), b2 + 1)
            else:
                wait_s(rb + j2 - 1, _o(0), b2)
                wait_s(rb + j2 - 1, _o(1), b2 + 1)
            if not last:
                if j2 == 7:
                    gather(rbn, _o(0), b2)
                    gather(rbn, _o(1), b2 + 1)
                else:
                    gather(rb + j2 + 1, _o(0), b2)
                    gather(rb + j2 + 1, _o(1), b2 + 1)
            wait_g(rb + j2, _o(0), b0)
            scat(rb + j2, _o(0), b0)
            wait_g(rb + j2, _o(1), b0 + 1)
            scat(rb + j2, _o(1), b0 + 1)

        gather(0, _o(0), 0)
        gather(0, _o(1), 1)
        gather(1, _o(0), 2)
        gather(1, _o(1), 3)
        wait_g(0, _o(0), 0)
        scat(0, _o(0), 0)
        wait_g(0, _o(1), 1)
        scat(0, _o(1), 1)
        load_chunk(1, 8)
        for j2 in range(1, 7):
            pair(0, None, 8, j2, last=False)
        wait_chunk(1, 8)
        pair(0, None, 8, 7, last=False)

        def chunk_body(cc, carry):
            rb = lax.rem(cc, 2) * 8
            rbn = lax.rem(cc + 1, 2) * 8
            pair(rb, rbn, rbn, 0, last=False)
            load_chunk(cc + 1, rbn)
            for j2 in range(1, 7):
                pair(rb, rbn, rbn, j2, last=False)
            wait_chunk(cc + 1, rbn)
            pair(rb, rbn, rbn, 7, last=False)
            return carry

        lax.fori_loop(1, nc - 1, chunk_body, 0)

        rb = ((nc - 1) % 2) * 8
        rbp = (nc % 2) * 8
        for j2 in range(7):
            pair(rb, rbp, rbp, j2, last=False)
        pair(rb, rbp, rbp, 7, last=True)
        wait_s(rb + 7, _o(0), 2)
        wait_s(rb + 7, _o(1), 3)

        plsc.subcore_barrier()

        pltpu.sync_copy(acc.at[pl.ds(s * rpt, rpt)],
                        out_hbm.at[c, pl.ds(s * rpt, rpt)])

        @pl.when(s == 0)
        def _out_tail():
            pltpu.sync_copy(acc.at[pl.ds(_NT * rpt, left)],
                            out_hbm.at[c, pl.ds(_NT * rpt, left)])

    return seg


_seg_full = _make_segsum(320)
_seg_half = _make_segsum(160)


def _window_idx(src, dst, n_win, table_rows):
    ept = src.shape[1] // _NT
    pad = n_win * _K - ept
    src_r = src.reshape(2, _NT, ept)
    dst_r = dst.reshape(2, _NT, ept)
    ar = jnp.arange(pad, dtype=jnp.int32)
    pad_src = jnp.broadcast_to(ar % table_rows, (2, _NT, pad))
    pad_dst = jnp.broadcast_to(N_NODES + ar % _JUNK, (2, _NT, pad))
    src_w = jnp.concatenate([src_r, pad_src], axis=2).reshape(-1, 2 * _K)
    dst_w = jnp.concatenate([dst_r, pad_dst], axis=2).reshape(-1, 2 * _K)
    return src_w, dst_w


_BP = 2000


def _proj_body(x_ref, w_ref, b_ref, o_ref):
    y = jnp.dot(x_ref[...], w_ref[0], preferred_element_type=jnp.float32)
    o_ref[...] = jnp.maximum(y + b_ref[0], 0.0)


def _proj(x_cat, w_stack, b_stack):
    n = x_cat.shape[0]
    grid = (n // _BP,)
    sel = lambda i: (i * _BP) // N_NODES
    return pl.pallas_call(
        _proj_body,
        grid=grid,
        in_specs=[
            pl.BlockSpec((_BP, D_IN), lambda i: (i, 0)),
            pl.BlockSpec((1, D_IN, HID), lambda i: (sel(i), 0, 0)),
            pl.BlockSpec((1, 1, HID), lambda i: (sel(i), 0, 0)),
        ],
        out_specs=pl.BlockSpec((_BP, HID), lambda i: (i, 0)),
        out_shape=jax.ShapeDtypeStruct((n, HID), jnp.float32),
    )(x_cat, w_stack, b_stack)


def _mlp2_body(a_ref, h_ref, w1_ref, b1_ref, w2_ref, b2_ref, o_ref):
    y = a_ref[...] + h_ref[...]
    t = jnp.dot(y, w1_ref[...], preferred_element_type=jnp.float32)
    t = jnp.maximum(t + b1_ref[...], 0.0)
    o = jnp.dot(t, w2_ref[...], preferred_element_type=jnp.float32)
    o_ref[...] = jnp.maximum(o + b2_ref[...], 0.0)


def _mlp2(agg, h_cat, off, w1, b1, w2, b2):
    grid = (N_NODES // _BP,)
    return pl.pallas_call(
        _mlp2_body,
        grid=grid,
        in_specs=[
            pl.BlockSpec((_BP, HID), lambda i: (i + off, 0)),
            pl.BlockSpec((_BP, HID), lambda i: (i + off, 0)),
            pl.BlockSpec((HID, HID), lambda i: (0, 0)),
            pl.BlockSpec((1, HID), lambda i: (0, 0)),
            pl.BlockSpec((HID, HID), lambda i: (0, 0)),
            pl.BlockSpec((1, HID), lambda i: (0, 0)),
        ],
        out_specs=pl.BlockSpec((_BP, HID), lambda i: (i, 0)),
        out_shape=jax.ShapeDtypeStruct((N_NODES, HID), jnp.float32),
    )(agg, h_cat, w1, b1, w2, b2)


def _final_body(a_ref, h_ref, p_ref, w1_ref, b1_ref, w2_ref, b2_ref,
                v1_ref, c1_ref, v2_ref, c2_ref, wo_ref, bo_ref, o_ref):
    y = a_ref[...] + h_ref[...]
    t = jnp.dot(y, w1_ref[...], preferred_element_type=jnp.float32)
    t = jnp.maximum(t + b1_ref[...], 0.0)
    hp = jnp.dot(t, w2_ref[...], preferred_element_type=jnp.float32)
    hp = jnp.maximum(hp + b2_ref[...], 0.0)
    y1 = p_ref[0] + p_ref[1] + hp
    t1 = jnp.dot(y1, v1_ref[...], preferred_element_type=jnp.float32)
    t1 = jnp.maximum(t1 + c1_ref[...], 0.0)
    h2 = jnp.dot(t1, v2_ref[...], preferred_element_type=jnp.float32)
    h2 = jnp.maximum(h2 + c2_ref[...], 0.0)
    o_ref[...] = jnp.dot(h2, wo_ref[...],
                         preferred_element_type=jnp.float32) + bo_ref[...]


def _final(agg_p, h_cat, parts, w1, b1, w2, b2, v1, c1, v2, c2, wo_pad,
           bo_pad):
    grid = (N_NODES // _BP,)
    wspec = pl.BlockSpec((HID, HID), lambda i: (0, 0))
    bspec = pl.BlockSpec((1, HID), lambda i: (0, 0))
    return pl.pallas_call(
        _final_body,
        grid=grid,
        in_specs=[
            pl.BlockSpec((_BP, HID), lambda i: (i, 0)),
            pl.BlockSpec((_BP, HID), lambda i: (i, 0)),
            pl.BlockSpec((2, _BP, HID), lambda i: (0, i, 0)),
            wspec, bspec, wspec, bspec,
            wspec, bspec, wspec, bspec,
            wspec, bspec,
        ],
        out_specs=pl.BlockSpec((_BP, HID), lambda i: (i, 0)),
        out_shape=jax.ShapeDtypeStruct((N_NODES, HID), jnp.float32),
    )(agg_p, h_cat, parts, w1, b1, w2, b2, v1, c1, v2, c2, wo_pad, bo_pad)


@jax.jit
def kernel(x_product, x_user, ei_buys, ei_rev, Wp_in, bp_in, Wu_in, bu_in,
           l0_buys_W1, l0_buys_b1, l0_buys_W2, l0_buys_b2,
           l0_rev_W1, l0_rev_b1, l0_rev_W2, l0_rev_b2,
           l1_buys_W1, l1_buys_b1, l1_buys_W2, l1_buys_b2,
           l1_rev_W1, l1_rev_b1, l1_rev_W2, l1_rev_b2,
           W_out, b_out):
    ei_b = jnp.asarray(ei_buys, jnp.int32)
    ei_r = jnp.asarray(ei_rev, jnp.int32)

    x_cat = jnp.concatenate([x_product, x_user], axis=0)
    w_in = jnp.stack([Wp_in, Wu_in])
    b_in = jnp.stack([bp_in, bu_in]).reshape(2, 1, HID)
    h_cat0 = _proj(x_cat, w_in, b_in)

    src0 = jnp.stack([ei_b[0] + N_NODES, ei_r[0]])
    dst0 = jnp.stack([ei_b[1], ei_r[1]])
    src0w, dst0w = _window_idx(src0, dst0, 320, 2 * N_NODES)
    agg0 = _seg_full(h_cat0, src0w, dst0w)

    agg0_flat = agg0.reshape(2 * N_NODES, HID)
    h_u1 = _mlp2(agg0_flat, h_cat0, N_NODES // _BP,
                 l0_rev_W1, l0_rev_b1.reshape(1, HID),
                 l0_rev_W2, l0_rev_b2.reshape(1, HID))

    src1w, dst1w = _window_idx(ei_b[0].reshape(2, EDGES // 2),
                               ei_b[1].reshape(2, EDGES // 2), 160, N_NODES)
    parts = _seg_half(h_u1, src1w, dst1w)

    wo_pad = jnp.zeros((HID, HID), jnp.float32).at[:, 0].set(W_out[:, 0])
    bo_pad = jnp.zeros((1, HID), jnp.float32).at[0, 0].set(b_out[0])
    out = _final(agg0_flat, h_cat0, parts,
                 l0_buys_W1, l0_buys_b1.reshape(1, HID),
                 l0_buys_W2, l0_buys_b2.reshape(1, HID),
                 l1_buys_W1, l1_buys_b1.reshape(1, HID),
                 l1_buys_W2, l1_buys_b2.reshape(1, HID), wo_pad, bo_pad)
    return out[:, 0]


# speedup vs baseline: 1.0575x; 1.0099x over previous
import functools

import jax
import jax.numpy as jnp
from jax import lax
from jax.experimental import pallas as pl
from jax.experimental.pallas import tpu as pltpu
from jax.experimental.pallas import tpu_sc as plsc

N_NODES = 10000
HID = 128
D_IN = 256
EDGES = 320000
_K = 64
_NT = 16
_ACC_N = 10240
_JUNK = _ACC_N - N_NODES
_CW = 16


def _make_segsum(n_win: int):
    nc = n_win // _CW
    assert n_win % _CW == 0 and nc >= 3
    rpt = 624
    left = N_NODES - _NT * rpt

    mesh = plsc.VectorSubcoreMesh(core_axis_name="c", subcore_axis_name="s")

    @functools.partial(
        pl.kernel,
        out_type=jax.ShapeDtypeStruct((2, N_NODES, HID), jnp.float32),
        mesh=mesh,
        scratch_types=[
            pltpu.VMEM_SHARED((_ACC_N, HID), jnp.float32),
            pltpu.VMEM((16, 2 * _K), jnp.int32),
            pltpu.VMEM((16, 2 * _K), jnp.int32),
            pltpu.VMEM((_K, HID), jnp.float32),
            pltpu.VMEM((_K, HID), jnp.float32),
            pltpu.VMEM((_K, HID), jnp.float32),
            pltpu.VMEM((_K, HID), jnp.float32),
            pltpu.SemaphoreType.DMA,
            pltpu.SemaphoreType.DMA,
            pltpu.SemaphoreType.DMA,
            pltpu.SemaphoreType.DMA,
            pltpu.SemaphoreType.DMA,
            pltpu.SemaphoreType.DMA,
            pltpu.SemaphoreType.DMA,
            pltpu.SemaphoreType.DMA,
            pltpu.SemaphoreType.DMA,
            pltpu.SemaphoreType.DMA,
        ],
    )
    def seg(table_hbm, src_hbm, dst_hbm, out_hbm, acc, idx_s, idx_d,
            r0, r1, r2, r3, g0, g1, g2, g3, s0, s1, s2, s3, li_s, li_d):
        c = lax.axis_index("c")
        s = lax.axis_index("s")
        rows = [r0, r1, r2, r3]
        semg = [g0, g1, g2, g3]
        sems = [s0, s1, s2, s3]
        base = (c * _NT + s) * (n_win // 2)

        def load_chunk(k, rb):
            pltpu.async_copy(src_hbm.at[pl.ds(base + k * 8, 8)],
                             idx_s.at[pl.ds(rb, 8)], li_s)
            pltpu.async_copy(dst_hbm.at[pl.ds(base + k * 8, 8)],
                             idx_d.at[pl.ds(rb, 8)], li_d)

        def wait_chunk(k, rb):
            pltpu.make_async_copy(src_hbm.at[pl.ds(base + k * 8, 8)],
                                  idx_s.at[pl.ds(rb, 8)], li_s).wait()
            pltpu.make_async_copy(dst_hbm.at[pl.ds(base + k * 8, 8)],
                                  idx_d.at[pl.ds(rb, 8)], li_d).wait()

        load_chunk(0, 0)
        zero = jnp.zeros((16,), jnp.float32)

        def zrow(r, carry):
            for k in range(HID // 16):
                r0[r, pl.ds(k * 16, 16)] = zero
            return carry

        lax.fori_loop(0, _K, zrow, 0)
        for j in range(rpt // _K):
            pltpu.sync_copy(r0, acc.at[pl.ds(s * rpt + j * _K, _K)])
        pltpu.sync_copy(r0.at[pl.ds(0, rpt % _K)],
                        acc.at[pl.ds(s * rpt + (rpt // _K) * _K, rpt % _K)])

        @pl.when(s == 0)
        def _zero_tail():
            pltpu.sync_copy(r0.at[pl.ds(0, left)],
                            acc.at[pl.ds(_NT * rpt, left)])

        wait_chunk(0, 0)
        plsc.subcore_barrier()

        def gather(r, o, b):
            return pltpu.async_copy(table_hbm.at[idx_s.at[r, pl.ds(o, _K)]],
                                    rows[b], semg[b])

        def scat(r, o, b):
            return pltpu.async_copy(rows[b], acc.at[idx_d.at[r, pl.ds(o, _K)]],
                                    sems[b], add=True)

        def wait_g(r, o, b):
            pltpu.make_async_copy(table_hbm.at[idx_s.at[r, pl.ds(o, _K)]],
                                  rows[b], semg[b]).wait()

        def wait_s(r, o, b):
            pltpu.make_async_copy(rows[b], acc.at[idx_d.at[r, pl.ds(o, _K)]],
                                  sems[b]).wait()

        _o = lambda j: (j % 2) * _K

        def pair(rb, rbp, rbn, j2, last):
            b0 = (2 * j2) % 4
            b2 = (b0 + 2) % 4
            if j2 == 0:
                wait_s(rbp + 7, _o(0), b2)
                wait_s(rbp + 7, _o(1), b2 + 1)
            else:
                wait_s(rb + j2 - 1, _o(0), b2)
                wait_s(rb + j2 - 1, _o(1), b2 + 1)
            if not last:
                if j2 == 7:
                    gather(rbn, _o(0), b2)
                    gather(rbn, _o(1), b2 + 1)
                else:
                    gather(rb + j2 + 1, _o(0), b2)
                    gather(rb + j2 + 1, _o(1), b2 + 1)
            wait_g(rb + j2, _o(0), b0)
            scat(rb + j2, _o(0), b0)
            wait_g(rb + j2, _o(1), b0 + 1)
            scat(rb + j2, _o(1), b0 + 1)

        gather(0, _o(0), 0)
        gather(0, _o(1), 1)
        gather(1, _o(0), 2)
        gather(1, _o(1), 3)
        wait_g(0, _o(0), 0)
        scat(0, _o(0), 0)
        wait_g(0, _o(1), 1)
        scat(0, _o(1), 1)
        load_chunk(1, 8)
        for j2 in range(1, 7):
            pair(0, None, 8, j2, last=False)
        wait_chunk(1, 8)
        pair(0, None, 8, 7, last=False)

        def chunk_body(cc, carry):
            rb = lax.rem(cc, 2) * 8
            rbn = lax.rem(cc + 1, 2) * 8
            pair(rb, rbn, rbn, 0, last=False)
            load_chunk(cc + 1, rbn)
            for j2 in range(1, 7):
                pair(rb, rbn, rbn, j2, last=False)
            wait_chunk(cc + 1, rbn)
            pair(rb, rbn, rbn, 7, last=False)
            return carry

        lax.fori_loop(1, nc - 1, chunk_body, 0)

        rb = ((nc - 1) % 2) * 8
        rbp = (nc % 2) * 8
        for j2 in range(7):
            pair(rb, rbp, rbp, j2, last=False)
        pair(rb, rbp, rbp, 7, last=True)
        wait_s(rb + 7, _o(0), 2)
        wait_s(rb + 7, _o(1), 3)

        plsc.subcore_barrier()

        pltpu.sync_copy(acc.at[pl.ds(s * rpt, rpt)],
                        out_hbm.at[c, pl.ds(s * rpt, rpt)])

        @pl.when(s == 0)
        def _out_tail():
            pltpu.sync_copy(acc.at[pl.ds(_NT * rpt, left)],
                            out_hbm.at[c, pl.ds(_NT * rpt, left)])

    return seg


_seg_full = _make_segsum(320)
_seg_half = _make_segsum(160)


def _window_idx(src, dst, n_win, table_rows):
    ept = src.shape[1] // _NT
    pad = n_win * _K - ept
    src_r = src.reshape(2, _NT, ept)
    dst_r = dst.reshape(2, _NT, ept)
    ar = jnp.arange(pad, dtype=jnp.int32)
    pad_src = jnp.broadcast_to(ar % table_rows, (2, _NT, pad))
    pad_dst = jnp.broadcast_to(N_NODES + ar % _JUNK, (2, _NT, pad))
    src_w = jnp.concatenate([src_r, pad_src], axis=2).reshape(-1, 2 * _K)
    dst_w = jnp.concatenate([dst_r, pad_dst], axis=2).reshape(-1, 2 * _K)
    return src_w, dst_w


_BP = 5000


def _proj_body(x_ref, w_ref, b_ref, o_ref):
    y = jnp.dot(x_ref[...], w_ref[0], preferred_element_type=jnp.float32)
    o_ref[...] = jnp.maximum(y + b_ref[0], 0.0)


def _proj(x_cat, w_stack, b_stack):
    n = x_cat.shape[0]
    grid = (n // _BP,)
    sel = lambda i: (i * _BP) // N_NODES
    return pl.pallas_call(
        _proj_body,
        grid=grid,
        in_specs=[
            pl.BlockSpec((_BP, D_IN), lambda i: (i, 0)),
            pl.BlockSpec((1, D_IN, HID), lambda i: (sel(i), 0, 0)),
            pl.BlockSpec((1, 1, HID), lambda i: (sel(i), 0, 0)),
        ],
        out_specs=pl.BlockSpec((_BP, HID), lambda i: (i, 0)),
        out_shape=jax.ShapeDtypeStruct((n, HID), jnp.float32),
    )(x_cat, w_stack, b_stack)


def _mlp2_body(a_ref, h_ref, w1_ref, b1_ref, w2_ref, b2_ref, o_ref):
    y = a_ref[...] + h_ref[...]
    t = jnp.dot(y, w1_ref[...], preferred_element_type=jnp.float32)
    t = jnp.maximum(t + b1_ref[...], 0.0)
    o = jnp.dot(t, w2_ref[...], preferred_element_type=jnp.float32)
    o_ref[...] = jnp.maximum(o + b2_ref[...], 0.0)


def _mlp2(agg, h_cat, off, w1, b1, w2, b2):
    grid = (N_NODES // _BP,)
    return pl.pallas_call(
        _mlp2_body,
        grid=grid,
        in_specs=[
            pl.BlockSpec((_BP, HID), lambda i: (i + off, 0)),
            pl.BlockSpec((_BP, HID), lambda i: (i + off, 0)),
            pl.BlockSpec((HID, HID), lambda i: (0, 0)),
            pl.BlockSpec((1, HID), lambda i: (0, 0)),
            pl.BlockSpec((HID, HID), lambda i: (0, 0)),
            pl.BlockSpec((1, HID), lambda i: (0, 0)),
        ],
        out_specs=pl.BlockSpec((_BP, HID), lambda i: (i, 0)),
        out_shape=jax.ShapeDtypeStruct((N_NODES, HID), jnp.float32),
    )(agg, h_cat, w1, b1, w2, b2)


def _final_body(a_ref, h_ref, p_ref, w1_ref, b1_ref, w2_ref, b2_ref,
                v1_ref, c1_ref, v2_ref, c2_ref, wo_ref, bo_ref, o_ref):
    y = a_ref[...] + h_ref[...]
    t = jnp.dot(y, w1_ref[...], preferred_element_type=jnp.float32)
    t = jnp.maximum(t + b1_ref[...], 0.0)
    hp = jnp.dot(t, w2_ref[...], preferred_element_type=jnp.float32)
    hp = jnp.maximum(hp + b2_ref[...], 0.0)
    y1 = p_ref[0] + p_ref[1] + hp
    t1 = jnp.dot(y1, v1_ref[...], preferred_element_type=jnp.float32)
    t1 = jnp.maximum(t1 + c1_ref[...], 0.0)
    h2 = jnp.dot(t1, v2_ref[...], preferred_element_type=jnp.float32)
    h2 = jnp.maximum(h2 + c2_ref[...], 0.0)
    o_ref[...] = jnp.dot(h2, wo_ref[...],
                         preferred_element_type=jnp.float32) + bo_ref[...]


def _final(agg_p, h_cat, parts, w1, b1, w2, b2, v1, c1, v2, c2, wo_pad,
           bo_pad):
    grid = (N_NODES // _BP,)
    wspec = pl.BlockSpec((HID, HID), lambda i: (0, 0))
    bspec = pl.BlockSpec((1, HID), lambda i: (0, 0))
    return pl.pallas_call(
        _final_body,
        grid=grid,
        in_specs=[
            pl.BlockSpec((_BP, HID), lambda i: (i, 0)),
            pl.BlockSpec((_BP, HID), lambda i: (i, 0)),
            pl.BlockSpec((2, _BP, HID), lambda i: (0, i, 0)),
            wspec, bspec, wspec, bspec,
            wspec, bspec, wspec, bspec,
            wspec, bspec,
        ],
        out_specs=pl.BlockSpec((_BP, HID), lambda i: (i, 0)),
        out_shape=jax.ShapeDtypeStruct((N_NODES, HID), jnp.float32),
    )(agg_p, h_cat, parts, w1, b1, w2, b2, v1, c1, v2, c2, wo_pad, bo_pad)


@jax.jit
def kernel(x_product, x_user, ei_buys, ei_rev, Wp_in, bp_in, Wu_in, bu_in,
           l0_buys_W1, l0_buys_b1, l0_buys_W2, l0_buys_b2,
           l0_rev_W1, l0_rev_b1, l0_rev_W2, l0_rev_b2,
           l1_buys_W1, l1_buys_b1, l1_buys_W2, l1_buys_b2,
           l1_rev_W1, l1_rev_b1, l1_rev_W2, l1_rev_b2,
           W_out, b_out):
    ei_b = jnp.asarray(ei_buys, jnp.int32)
    ei_r = jnp.asarray(ei_rev, jnp.int32)

    x_cat = jnp.concatenate([x_product, x_user], axis=0)
    w_in = jnp.stack([Wp_in, Wu_in])
    b_in = jnp.stack([bp_in, bu_in]).reshape(2, 1, HID)
    h_cat0 = _proj(x_cat, w_in, b_in)

    src0 = jnp.stack([ei_b[0] + N_NODES, ei_r[0]])
    dst0 = jnp.stack([ei_b[1], ei_r[1]])
    src0w, dst0w = _window_idx(src0, dst0, 320, 2 * N_NODES)
    agg0 = _seg_full(h_cat0, src0w, dst0w)

    agg0_flat = agg0.reshape(2 * N_NODES, HID)
    h_u1 = _mlp2(agg0_flat, h_cat0, N_NODES // _BP,
                 l0_rev_W1, l0_rev_b1.reshape(1, HID),
                 l0_rev_W2, l0_rev_b2.reshape(1, HID))

    src1w, dst1w = _window_idx(ei_b[0].reshape(2, EDGES // 2),
                               ei_b[1].reshape(2, EDGES // 2), 160, N_NODES)
    parts = _seg_half(h_u1, src1w, dst1w)

    wo_pad = jnp.zeros((HID, HID), jnp.float32).at[:, 0].set(W_out[:, 0])
    bo_pad = jnp.zeros((1, HID), jnp.float32).at[0, 0].set(b_out[0])
    out = _final(agg0_flat, h_cat0, parts,
                 l0_buys_W1, l0_buys_b1.reshape(1, HID),
                 l0_buys_W2, l0_buys_b2.reshape(1, HID),
                 l1_buys_W1, l1_buys_b1.reshape(1, HID),
                 l1_buys_W2, l1_buys_b2.reshape(1, HID), wo_pad, bo_pad)
    return out[:, 0]
